# baseline (speedup 1.0000x reference)
import functools

import jax
import jax.numpy as jnp
from jax import lax
from jax.experimental import pallas as pl
from jax.experimental.pallas import tpu as pltpu
from jax.experimental.pallas import tpu_sc as plsc

EPS = 1e-10
NUM_CORES = 2
NUM_SUBCORES = 16
LANES = 16
NW = NUM_CORES * NUM_SUBCORES

_LOG_POLY = (
    7.0376836292e-2,
    -1.1514610310e-1,
    1.1676998740e-1,
    -1.2420140846e-1,
    1.4249322787e-1,
    -1.6668057665e-1,
    2.0000714765e-1,
    -2.4999993993e-1,
    3.3333331174e-1,
)


def _vlog(x):
    bits = lax.bitcast_convert_type(x, jnp.int32)
    ex = (bits >> 23) - 127
    mbits = (bits & jnp.int32(0x007FFFFF)) | jnp.int32(0x3F800000)
    m = lax.bitcast_convert_type(mbits, jnp.float32)
    f = 0.5 * m
    e = ex + 1
    small = f < 0.70710678
    e = jnp.where(small, e - 1, e).astype(jnp.float32)
    xf = jnp.where(small, f + f - 1.0, f - 1.0)
    z = xf * xf
    acc = jnp.full((LANES,), _LOG_POLY[0], jnp.float32)
    for c in _LOG_POLY[1:]:
        acc = acc * xf + c
    y = xf * (z * acc)
    y = y + e * (-2.12194440e-4)
    y = y - 0.5 * z
    y = y + xf
    y = y + e * 0.693359375
    return y


@functools.lru_cache(maxsize=None)
def _build_sc_sampler(B, V):
    assert V % LANES == 0 and B % LANES == 0
    WN = ((V + NW * LANES - 1) // (NW * LANES)) * LANES
    LAST = V - (NW - 1) * WN
    assert 0 < LAST <= WN and LAST % LANES == 0

    mesh = plsc.VectorSubcoreMesh(
        core_axis_name="c", subcore_axis_name="s",
        num_cores=NUM_CORES, num_subcores=NUM_SUBCORES)

    @functools.partial(
        pl.kernel,
        mesh=mesh,
        compiler_params=pltpu.CompilerParams(needs_layout_passes=False),
        out_type=[
            jax.ShapeDtypeStruct((NW * B,), jnp.float32),
            jax.ShapeDtypeStruct((NW * B,), jnp.int32),
        ],
        scratch_types=[
            pltpu.VMEM((WN,), jnp.float32),
            pltpu.VMEM((2 * WN,), jnp.float32),
            pltpu.VMEM((B,), jnp.float32),
            pltpu.VMEM((B,), jnp.float32),
            pltpu.VMEM((B,), jnp.int32),
            pltpu.SemaphoreType.DMA,
            pltpu.SemaphoreType.DMA,
        ],
    )
    def sc_sampler(logits_hbm, temps_hbm, exp_hbm, pvals_hbm, pidx_hbm,
                   nbuf, lbuf, tbuf, vbuf, ibuf, sem0, sem1):
        cid = lax.axis_index("c")
        sid = lax.axis_index("s")
        wid = sid * NUM_CORES + cid
        pltpu.sync_copy(temps_hbm, tbuf)
        iota = lax.iota(jnp.int32, LANES)

        def work(start, size):
            sems = (sem0, sem1)

            def row_copy(b):
                return pltpu.make_async_copy(
                    logits_hbm.at[pl.ds(b * V + start, size)],
                    lbuf.at[pl.ds((b % 2) * WN, size)],
                    sems[b % 2])

            row_copy(0).start()
            pltpu.sync_copy(exp_hbm.at[pl.ds(start, size)],
                            nbuf.at[pl.ds(0, size)])
            nsteps = size // LANES

            def logstep(j, carry):
                sl = pl.ds(j * LANES, LANES)
                nbuf[sl] = _vlog(jnp.maximum(nbuf[sl], EPS))
                return carry

            lax.fori_loop(0, nsteps, logstep, 0)

            resv = jnp.zeros((LANES,), jnp.float32)
            resi = jnp.zeros((LANES,), jnp.int32)
            for b in range(B):
                row_copy(b).wait()
                if b + 1 < B:
                    row_copy(b + 1).start()
                tg = tbuf[pl.ds((b // LANES) * LANES, LANES)]
                t = tg[b % LANES]
                a = jnp.where(t >= EPS, t, 0.0)
                lbase = (b % 2) * WN

                def step(j, carry, lbase=lbase, a=a):
                    vmax, vidx, cur = carry
                    sl = pl.ds(j * LANES, LANES)
                    kv = lbuf[pl.ds(lbase + j * LANES, LANES)] - a * nbuf[sl]
                    m = kv > vmax
                    return (jnp.where(m, kv, vmax),
                            jnp.where(m, cur, vidx),
                            cur + LANES)

                vmax0 = jnp.full((LANES,), -jnp.inf, jnp.float32)
                vidx0 = jnp.zeros((LANES,), jnp.int32)
                cur0 = start + iota
                vmax, vidx, _ = lax.fori_loop(
                    0, nsteps, step, (vmax0, vidx0, cur0))
                mv = jnp.max(vmax)
                best = jnp.min(jnp.where(vmax == mv, vidx,
                                         jnp.int32(2147483647)))
                lane = iota == (b % LANES)
                resv = jnp.where(lane, mv, resv)
                resi = jnp.where(lane, best, resi)
                if b % LANES == LANES - 1:
                    g = (b // LANES) * LANES
                    vbuf[pl.ds(g, LANES)] = resv
                    ibuf[pl.ds(g, LANES)] = resi

        if LAST == WN:
            work(wid * WN, WN)
        else:
            @pl.when(wid != NW - 1)
            def _full():
                work(wid * WN, WN)

            @pl.when(wid == NW - 1)
            def _tail():
                work((NW - 1) * WN, LAST)

        pltpu.sync_copy(vbuf, pvals_hbm.at[pl.ds(wid * B, B)])
        pltpu.sync_copy(ibuf, pidx_hbm.at[pl.ds(wid * B, B)])

    return sc_sampler


def kernel(logits, temperatures, exponential):
    B, V = logits.shape
    pv, pi = _build_sc_sampler(B, V)(
        logits.reshape(B * V), temperatures, exponential.reshape(V))
    pv = pv.reshape(NW, B)
    pi = pi.reshape(NW, B)
    w = jnp.argmax(pv, axis=0)
    out = jnp.take_along_axis(pi, w[None, :], axis=0)[0]
    return out.astype(jnp.int32)


# speedup vs baseline: 1.0738x; 1.0738x over previous
import functools

import jax
import jax.numpy as jnp
from jax import lax
from jax.experimental import pallas as pl
from jax.experimental.pallas import tpu as pltpu
from jax.experimental.pallas import tpu_sc as plsc

EPS = 1e-10
NUM_CORES = 2
NUM_SUBCORES = 16
LANES = 16
NW = NUM_CORES * NUM_SUBCORES
U = 8
NEG_INF = float("-inf")
INT_MAX = 2147483647

_LOG_POLY = (
    7.0376836292e-2,
    -1.1514610310e-1,
    1.1676998740e-1,
    -1.2420140846e-1,
    1.4249322787e-1,
    -1.6668057665e-1,
    2.0000714765e-1,
    -2.4999993993e-1,
    3.3333331174e-1,
)


def _vlog(x):
    bits = lax.bitcast_convert_type(x, jnp.int32)
    ex = (bits >> 23) - 127
    mbits = (bits & jnp.int32(0x007FFFFF)) | jnp.int32(0x3F800000)
    m = lax.bitcast_convert_type(mbits, jnp.float32)
    f = 0.5 * m
    e = ex + 1
    small = f < 0.70710678
    e = jnp.where(small, e - 1, e).astype(jnp.float32)
    xf = jnp.where(small, f + f - 1.0, f - 1.0)
    z = xf * xf
    acc = jnp.full((LANES,), _LOG_POLY[0], jnp.float32)
    for c in _LOG_POLY[1:]:
        acc = acc * xf + c
    y = xf * (z * acc)
    y = y + e * (-2.12194440e-4)
    y = y - 0.5 * z
    y = y + xf
    y = y + e * 0.693359375
    return y


def _merge(cand):
    while len(cand) > 1:
        nxt = []
        for i in range(0, len(cand) - 1, 2):
            av, ai = cand[i]
            bv, bi = cand[i + 1]
            takeb = (bv > av) | ((bv == av) & (bi < ai))
            nxt.append((jnp.where(takeb, bv, av), jnp.where(takeb, bi, ai)))
        if len(cand) % 2:
            nxt.append(cand[-1])
        cand = nxt
    return cand[0]


@functools.lru_cache(maxsize=None)
def _build_sc_sampler(B, V):
    assert V % LANES == 0 and B % LANES == 0 and B % 2 == 0
    WN = ((V + NW * LANES - 1) // (NW * LANES)) * LANES
    LAST = V - (NW - 1) * WN
    assert 0 < LAST <= WN and LAST % LANES == 0

    mesh = plsc.VectorSubcoreMesh(
        core_axis_name="c", subcore_axis_name="s",
        num_cores=NUM_CORES, num_subcores=NUM_SUBCORES)

    @functools.partial(
        pl.kernel,
        mesh=mesh,
        compiler_params=pltpu.CompilerParams(needs_layout_passes=False),
        out_type=[
            jax.ShapeDtypeStruct((NW * B,), jnp.float32),
            jax.ShapeDtypeStruct((NW * B,), jnp.int32),
        ],
        scratch_types=[
            pltpu.VMEM((WN,), jnp.float32),
            pltpu.VMEM((2 * WN,), jnp.float32),
            pltpu.VMEM((B,), jnp.float32),
            pltpu.VMEM((B,), jnp.float32),
            pltpu.VMEM((B,), jnp.int32),
            pltpu.SemaphoreType.DMA,
            pltpu.SemaphoreType.DMA,
        ],
    )
    def sc_sampler(logits_hbm, temps_hbm, exp_hbm, pvals_hbm, pidx_hbm,
                   nbuf, lbuf, tbuf, vbuf, ibuf, sem0, sem1):
        cid = lax.axis_index("c")
        sid = lax.axis_index("s")
        wid = sid * NUM_CORES + cid
        pltpu.sync_copy(temps_hbm, tbuf)
        iota = lax.iota(jnp.int32, LANES)
        sems = (sem0, sem1)

        def work(start, size):
            nsteps = size // LANES
            trips = nsteps // U
            rem = nsteps - trips * U

            def row_copy(b, parity):
                return pltpu.make_async_copy(
                    logits_hbm.at[pl.ds(b * V + start, size)],
                    lbuf.at[pl.ds(parity * WN, size)],
                    sems[parity])

            row_copy(0, 0).start()
            row_copy(1, 1).start()
            pltpu.sync_copy(exp_hbm.at[pl.ds(start, size)],
                            nbuf.at[pl.ds(0, size)])

            def logstep(j, carry):
                base = j * (U * LANES)
                for g in range(U):
                    sl = pl.ds(base + g * LANES, LANES)
                    nbuf[sl] = _vlog(jnp.maximum(nbuf[sl], EPS))
                return carry

            lax.fori_loop(0, trips, logstep, 0)
            for r in range(rem):
                sl = pl.ds((trips * U + r) * LANES, LANES)
                nbuf[sl] = _vlog(jnp.maximum(nbuf[sl], EPS))

            tg0 = tbuf[pl.ds(0, LANES)]
            tg1 = tbuf[pl.ds(LANES, LANES)]
            sv = start + iota
            neg = jnp.full((LANES,), NEG_INF, jnp.float32)
            zero = jnp.zeros((LANES,), jnp.int32)

            def rowpair(bb, carry):
                resv0, resv1, resi0, resi1 = carry
                for db in range(2):
                    b = bb * 2 + db
                    row_copy(b, db).wait()
                    is_lo = b < LANES
                    tsel = jnp.where(is_lo, tg0, tg1)
                    lm = iota == (b & (LANES - 1))
                    t = jnp.max(jnp.where(lm, tsel, NEG_INF))
                    a = jnp.where(t >= EPS, t, 0.0)
                    lbase = db * WN

                    def step(j, c, a=a, lbase=lbase):
                        accs = list(c)
                        jd = jnp.full((LANES,), 0, jnp.int32) + j
                        base = j * (U * LANES)
                        for g in range(U):
                            lv = lbuf[pl.ds(lbase + base + g * LANES, LANES)]
                            nv = nbuf[pl.ds(base + g * LANES, LANES)]
                            kv = lv - a * nv
                            vm, vj = accs[2 * g], accs[2 * g + 1]
                            m = kv > vm
                            accs[2 * g] = jnp.where(m, kv, vm)
                            accs[2 * g + 1] = jnp.where(m, jd, vj)
                        return tuple(accs)

                    init = []
                    for g in range(U):
                        init += [neg, zero]
                    accs = list(lax.fori_loop(0, trips, step, tuple(init)))
                    jd = jnp.full((LANES,), trips, jnp.int32)
                    for r in range(rem):
                        off = (trips * U + r) * LANES
                        lv = lbuf[pl.ds(lbase + off, LANES)]
                        nv = nbuf[pl.ds(off, LANES)]
                        kv = lv - a * nv
                        vm, vj = accs[2 * r], accs[2 * r + 1]
                        m = kv > vm
                        accs[2 * r] = jnp.where(m, kv, vm)
                        accs[2 * r + 1] = jnp.where(m, jd, vj)
                    cand = []
                    for g in range(U):
                        idx = ((accs[2 * g + 1] * U + g) * LANES) + sv
                        cand.append((accs[2 * g], idx))
                    val, idx = _merge(cand)
                    mv = jnp.max(val)
                    best = jnp.min(jnp.where(val == mv, idx,
                                             jnp.int32(INT_MAX)))
                    mask0 = lm & is_lo
                    mask1 = lm & (~is_lo)
                    resv0 = jnp.where(mask0, mv, resv0)
                    resi0 = jnp.where(mask0, best, resi0)
                    resv1 = jnp.where(mask1, mv, resv1)
                    resi1 = jnp.where(mask1, best, resi1)

                    @pl.when(b + 2 < B)
                    def _next(b=b, db=db):
                        row_copy(b + 2, db).start()
                return resv0, resv1, resi0, resi1

            resv0, resv1, resi0, resi1 = lax.fori_loop(
                0, B // 2, rowpair, (neg, neg, zero, zero))
            vbuf[pl.ds(0, LANES)] = resv0
            vbuf[pl.ds(LANES, LANES)] = resv1
            ibuf[pl.ds(0, LANES)] = resi0
            ibuf[pl.ds(LANES, LANES)] = resi1

        if LAST == WN:
            work(wid * WN, WN)
        else:
            @pl.when(wid != NW - 1)
            def _full():
                work(wid * WN, WN)

            @pl.when(wid == NW - 1)
            def _tail():
                work((NW - 1) * WN, LAST)

        pltpu.sync_copy(vbuf, pvals_hbm.at[pl.ds(wid * B, B)])
        pltpu.sync_copy(ibuf, pidx_hbm.at[pl.ds(wid * B, B)])

    return sc_sampler


def kernel(logits, temperatures, exponential):
    B, V = logits.shape
    pv, pi = _build_sc_sampler(B, V)(
        logits.reshape(B * V), temperatures, exponential.reshape(V))
    pv = pv.reshape(NW, B)
    pi = pi.reshape(NW, B)
    w = jnp.argmax(pv, axis=0)
    out = jnp.take_along_axis(pi, w[None, :], axis=0)[0]
    return out.astype(jnp.int32)


# speedup vs baseline: 16.2384x; 15.1224x over previous
import functools

import jax
import jax.numpy as jnp
from jax import lax
from jax.experimental import pallas as pl
from jax.experimental.pallas import tpu as pltpu
from jax.experimental.pallas import tpu_sc as plsc

EPS = 1e-10
NUM_CORES = 2
NUM_SUBCORES = 16
LANES = 16
NW = NUM_CORES * NUM_SUBCORES
TILE = 128
KPT = TILE // LANES
NEG_INF = float("-inf")
INT_MAX = 2147483647

_LOG_POLY = (
    7.0376836292e-2,
    -1.1514610310e-1,
    1.1676998740e-1,
    -1.2420140846e-1,
    1.4249322787e-1,
    -1.6668057665e-1,
    2.0000714765e-1,
    -2.4999993993e-1,
    3.3333331174e-1,
)


def _vlog(x):
    bits = lax.bitcast_convert_type(x, jnp.int32)
    ex = (bits >> 23) - 127
    mbits = (bits & jnp.int32(0x007FFFFF)) | jnp.int32(0x3F800000)
    m = lax.bitcast_convert_type(mbits, jnp.float32)
    f = 0.5 * m
    e = ex + 1
    small = f < 0.70710678
    e = jnp.where(small, e - 1, e).astype(jnp.float32)
    xf = jnp.where(small, f + f - 1.0, f - 1.0)
    z = xf * xf
    acc = jnp.full((LANES,), _LOG_POLY[0], jnp.float32)
    for c in _LOG_POLY[1:]:
        acc = acc * xf + c
    y = xf * (z * acc)
    y = y + e * (-2.12194440e-4)
    y = y - 0.5 * z
    y = y + xf
    y = y + e * 0.693359375
    return y


def _merge(cand):
    while len(cand) > 1:
        nxt = []
        for i in range(0, len(cand) - 1, 2):
            av, ai = cand[i]
            bv, bi = cand[i + 1]
            takeb = (bv > av) | ((bv == av) & (bi < ai))
            nxt.append((jnp.where(takeb, bv, av), jnp.where(takeb, bi, ai)))
        if len(cand) % 2:
            nxt.append(cand[-1])
        cand = nxt
    return cand[0]


@functools.lru_cache(maxsize=None)
def _build_sc_sampler(B, V):
    assert B == 32
    NT = V // TILE
    TPW = -(-NT // NW)
    CT = 35
    TPW = -(-TPW // CT) * CT
    NCH = TPW // CT
    assert NCH >= 3 and TPW <= NT
    STEP = (NT - TPW) // (NW - 1)
    assert STEP <= TPW

    mesh = plsc.VectorSubcoreMesh(
        core_axis_name="c", subcore_axis_name="s",
        num_cores=NUM_CORES, num_subcores=NUM_SUBCORES)

    @functools.partial(
        pl.kernel,
        mesh=mesh,
        compiler_params=pltpu.CompilerParams(
            needs_layout_passes=False, use_tc_tiling_on_sc=True),
        out_type=[
            jax.ShapeDtypeStruct((NW * B,), jnp.float32),
            jax.ShapeDtypeStruct((NW * B,), jnp.int32),
        ],
        scratch_types=[
            pltpu.VMEM((TPW * TILE,), jnp.float32),
            pltpu.VMEM((8, CT * TILE), jnp.float32),
            pltpu.VMEM((8, CT * TILE), jnp.float32),
            pltpu.VMEM((B,), jnp.float32),
            pltpu.VMEM((B,), jnp.float32),
            pltpu.VMEM((B,), jnp.int32),
            pltpu.SemaphoreType.DMA,
            pltpu.SemaphoreType.DMA,
        ],
    )
    def sc_sampler(logits_hbm, temps_hbm, exp_hbm, pvals_hbm, pidx_hbm,
                   nbuf, lbufa, lbufb, tbuf, vbuf, ibuf, sema, semb):
        cid = lax.axis_index("c")
        sid = lax.axis_index("s")
        wid = sid * NUM_CORES + cid
        t0 = jnp.minimum(wid * STEP, NT - TPW)
        pltpu.sync_copy(temps_hbm, tbuf)
        iota = lax.iota(jnp.int32, LANES)
        neg = jnp.full((LANES,), NEG_INF, jnp.float32)
        zero = jnp.zeros((LANES,), jnp.int32)
        bufs = (lbufa, lbufb)
        sems = (sema, semb)

        def chunk_copy(rg, ci, parity):
            return pltpu.make_async_copy(
                logits_hbm.at[pl.ds(rg * 8, 8),
                              pl.ds((t0 + ci * CT) * TILE, CT * TILE)],
                bufs[parity], sems[parity])

        chunk_copy(0, NCH - 1, 0).start()
        chunk_copy(0, 0, 1).start()
        pltpu.sync_copy(exp_hbm.at[pl.ds(t0 * TILE, TPW * TILE)], nbuf)

        def logstep(j, carry):
            base = j * TILE
            for g in range(KPT):
                sl = pl.ds(base + g * LANES, LANES)
                nbuf[sl] = _vlog(jnp.maximum(nbuf[sl], EPS))
            return carry

        lax.fori_loop(0, TPW, logstep, 0)

        tg0 = tbuf[pl.ds(0, LANES)]
        tg1 = tbuf[pl.ds(LANES, LANES)]

        def proc(buf, chunkbase, avals, rv, ri):
            nbase = chunkbase * TILE
            basev = ((t0 + chunkbase) * TILE) + iota
            rv, ri = list(rv), list(ri)
            for r in range(8):
                a = avals[r]

                def step(ct, c, a=a, r=r):
                    accs = list(c)
                    jd = zero + ct
                    coff = ct * TILE
                    for k in range(KPT):
                        lv = buf[r, pl.ds(coff + k * LANES, LANES)]
                        nv = nbuf[pl.ds(nbase + coff + k * LANES, LANES)]
                        kv = lv - a * nv
                        vm, vj = accs[2 * k], accs[2 * k + 1]
                        m = kv > vm
                        accs[2 * k] = jnp.where(m, kv, vm)
                        accs[2 * k + 1] = jnp.where(m, jd, vj)
                    return tuple(accs)

                init = []
                for k in range(KPT):
                    init += [neg, zero]
                accs = lax.fori_loop(0, CT, step, tuple(init))
                cand = []
                for k in range(KPT):
                    idx = (accs[2 * k + 1] * TILE) + (basev + k * LANES)
                    cand.append((accs[2 * k], idx))
                val, idx = _merge(cand)
                rv[r], ri[r] = _merge([(rv[r], ri[r]), (val, idx)])
            return rv, ri

        def rowgroup(rg, carry):
            resv0, resv1, resi0, resi1 = carry
            avals = []
            for r in range(8):
                b = rg * 8 + r
                tsel = jnp.where(b < 16, tg0, tg1)
                lm = iota == (b & (2 * LANES - 1)) % LANES
                t = jnp.max(jnp.where(lm, tsel, NEG_INF))
                avals.append(jnp.where(t >= EPS, t, 0.0))
            rv = [neg] * 8
            ri = [zero] * 8
            chunk_copy(rg, NCH - 1, 0).wait()
            rv, ri = proc(lbufa, (NCH - 1) * CT, avals, rv, ri)
            chunk_copy(rg, 1, 0).start()

            def trips(tt, c):
                rvri = list(c)
                rv, ri = rvri[:8], rvri[8:]
                ce = tt * 2
                chunk_copy(rg, ce, 1).wait()
                rv, ri = proc(lbufb, ce * CT, avals, rv, ri)

                @pl.when(ce + 2 <= NCH - 2)
                def _sb():
                    chunk_copy(rg, ce + 2, 1).start()

                co = tt * 2 + 1
                chunk_copy(rg, co, 0).wait()
                rv, ri = proc(lbufa, co * CT, avals, rv, ri)

                @pl.when(co + 2 <= NCH - 2)
                def _sa():
                    chunk_copy(rg, co + 2, 0).start()

                return tuple(rv + ri)

            assert (NCH - 1) % 2 == 0
            out = list(lax.fori_loop(0, (NCH - 1) // 2, trips,
                                     tuple(rv + ri)))
            rv, ri = out[:8], out[8:]

            @pl.when(rg < 3)
            def _prime():
                chunk_copy(rg + 1, NCH - 1, 0).start()
                chunk_copy(rg + 1, 0, 1).start()

            for r in range(8):
                b = rg * 8 + r
                mv = jnp.max(rv[r])
                best = jnp.min(jnp.where(rv[r] == mv, ri[r],
                                         jnp.int32(INT_MAX)))
                lm = iota == (b & (2 * LANES - 1)) % LANES
                lo = b < 16
                m0 = lm & lo
                m1 = lm & (~lo)
                resv0 = jnp.where(m0, mv, resv0)
                resi0 = jnp.where(m0, best, resi0)
                resv1 = jnp.where(m1, mv, resv1)
                resi1 = jnp.where(m1, best, resi1)
            return resv0, resv1, resi0, resi1

        resv0, resv1, resi0, resi1 = lax.fori_loop(
            0, 4, rowgroup, (neg, neg, zero, zero))
        vbuf[pl.ds(0, LANES)] = resv0
        vbuf[pl.ds(LANES, LANES)] = resv1
        ibuf[pl.ds(0, LANES)] = resi0
        ibuf[pl.ds(LANES, LANES)] = resi1
        pltpu.sync_copy(vbuf, pvals_hbm.at[pl.ds(wid * B, B)])
        pltpu.sync_copy(ibuf, pidx_hbm.at[pl.ds(wid * B, B)])

    return sc_sampler, NT * TILE


def kernel(logits, temperatures, exponential):
    B, V = logits.shape
    sampler, vcov = _build_sc_sampler(B, V)
    ev = exponential.reshape(V)
    pv, pi = sampler(logits, temperatures, ev)
    pv = pv.reshape(NW, B)
    pi = pi.reshape(NW, B)
    if vcov < V:
        a = jnp.where(temperatures >= EPS, temperatures, 0.0)
        ntail = jnp.log(jnp.maximum(ev[vcov:], EPS))
        tk = logits[:, vcov:] - a[:, None] * ntail
        tv = jnp.max(tk, axis=-1)
        ti = (vcov + jnp.argmax(tk, axis=-1)).astype(jnp.int32)
        pv = jnp.concatenate([pv, tv[None, :]], axis=0)
        pi = jnp.concatenate([pi, ti[None, :]], axis=0)
    w = jnp.argmax(pv, axis=0)
    out = jnp.take_along_axis(pi, w[None, :], axis=0)[0]
    return out.astype(jnp.int32)


# speedup vs baseline: 22.3790x; 1.3782x over previous
import functools

import jax
import jax.numpy as jnp
from jax import lax
from jax.experimental import pallas as pl
from jax.experimental.pallas import tpu as pltpu
from jax.experimental.pallas import tpu_sc as plsc

EPS = 1e-10
NUM_CORES = 2
NUM_SUBCORES = 16
LANES = 16
NW = NUM_CORES * NUM_SUBCORES
TILE = 128
KPT = TILE // LANES
NEG_INF = float("-inf")
INT_MAX = 2147483647

def _tc_log_noise(exponential):
    V = exponential.shape[1]
    C = 65536
    grid = -(-V // C)

    def body(e_ref, n_ref):
        n_ref[...] = jnp.log(jnp.maximum(e_ref[0, :], EPS))

    return pl.pallas_call(
        body,
        grid=(grid,),
        in_specs=[pl.BlockSpec((1, C), lambda i: (0, i))],
        out_specs=pl.BlockSpec((C,), lambda i: (i,)),
        out_shape=jax.ShapeDtypeStruct((V,), jnp.float32),
    )(exponential)


def _merge(cand):
    while len(cand) > 1:
        nxt = []
        for i in range(0, len(cand) - 1, 2):
            av, ai = cand[i]
            bv, bi = cand[i + 1]
            takeb = (bv > av) | ((bv == av) & (bi < ai))
            nxt.append((jnp.where(takeb, bv, av), jnp.where(takeb, bi, ai)))
        if len(cand) % 2:
            nxt.append(cand[-1])
        cand = nxt
    return cand[0]


@functools.lru_cache(maxsize=None)
def _build_sc_sampler(B, V):
    assert B == 32
    NT = V // TILE
    TPW = -(-NT // NW)
    CT = 35
    TPW = -(-TPW // CT) * CT
    NCH = TPW // CT
    assert NCH >= 3 and TPW <= NT
    STEP = (NT - TPW) // (NW - 1)
    assert STEP <= TPW

    mesh = plsc.VectorSubcoreMesh(
        core_axis_name="c", subcore_axis_name="s",
        num_cores=NUM_CORES, num_subcores=NUM_SUBCORES)

    @functools.partial(
        pl.kernel,
        mesh=mesh,
        compiler_params=pltpu.CompilerParams(
            needs_layout_passes=False, use_tc_tiling_on_sc=True),
        out_type=[
            jax.ShapeDtypeStruct((NW * B,), jnp.float32),
            jax.ShapeDtypeStruct((NW * B,), jnp.int32),
        ],
        scratch_types=[
            pltpu.VMEM((TPW * TILE,), jnp.float32),
            pltpu.VMEM((8, CT * TILE), jnp.float32),
            pltpu.VMEM((8, CT * TILE), jnp.float32),
            pltpu.VMEM((B,), jnp.float32),
            pltpu.VMEM((B,), jnp.float32),
            pltpu.VMEM((B,), jnp.int32),
            pltpu.SemaphoreType.DMA,
            pltpu.SemaphoreType.DMA,
        ],
    )
    def sc_sampler(logits_hbm, temps_hbm, noise_hbm, pvals_hbm, pidx_hbm,
                   nbuf, lbufa, lbufb, tbuf, vbuf, ibuf, sema, semb):
        cid = lax.axis_index("c")
        sid = lax.axis_index("s")
        wid = sid * NUM_CORES + cid
        t0 = jnp.minimum(wid * STEP, NT - TPW)
        pltpu.sync_copy(temps_hbm, tbuf)
        iota = lax.iota(jnp.int32, LANES)
        neg = jnp.full((LANES,), NEG_INF, jnp.float32)
        zero = jnp.zeros((LANES,), jnp.int32)
        bufs = (lbufa, lbufb)
        sems = (sema, semb)

        def chunk_copy(rg, ci, parity):
            return pltpu.make_async_copy(
                logits_hbm.at[pl.ds(rg * 8, 8),
                              pl.ds((t0 + ci * CT) * TILE, CT * TILE)],
                bufs[parity], sems[parity])

        chunk_copy(0, NCH - 1, 0).start()
        chunk_copy(0, 0, 1).start()
        pltpu.sync_copy(noise_hbm.at[pl.ds(t0 * TILE, TPW * TILE)], nbuf)

        tg0 = tbuf[pl.ds(0, LANES)]
        tg1 = tbuf[pl.ds(LANES, LANES)]

        def proc(buf, chunkbase, avals, rv, ri):
            nbase = chunkbase * TILE
            basev = ((t0 + chunkbase) * TILE) + iota
            rv, ri = list(rv), list(ri)
            for r in range(8):
                a = avals[r]

                def step(ct, c, a=a, r=r):
                    accs = list(c)
                    jd = zero + ct
                    coff = ct * TILE
                    for k in range(KPT):
                        lv = buf[r, pl.ds(coff + k * LANES, LANES)]
                        nv = nbuf[pl.ds(nbase + coff + k * LANES, LANES)]
                        kv = lv - a * nv
                        vm, vj = accs[2 * k], accs[2 * k + 1]
                        m = kv > vm
                        accs[2 * k] = jnp.where(m, kv, vm)
                        accs[2 * k + 1] = jnp.where(m, jd, vj)
                    return tuple(accs)

                init = []
                for k in range(KPT):
                    init += [neg, zero]
                accs = lax.fori_loop(0, CT, step, tuple(init))
                cand = []
                for k in range(KPT):
                    idx = (accs[2 * k + 1] * TILE) + (basev + k * LANES)
                    cand.append((accs[2 * k], idx))
                val, idx = _merge(cand)
                rv[r], ri[r] = _merge([(rv[r], ri[r]), (val, idx)])
            return rv, ri

        def rowgroup(rg, carry):
            resv0, resv1, resi0, resi1 = carry
            avals = []
            for r in range(8):
                b = rg * 8 + r
                tsel = jnp.where(b < 16, tg0, tg1)
                lm = iota == (b & (2 * LANES - 1)) % LANES
                t = jnp.max(jnp.where(lm, tsel, NEG_INF))
                avals.append(jnp.where(t >= EPS, t, 0.0))
            rv = [neg] * 8
            ri = [zero] * 8
            chunk_copy(rg, NCH - 1, 0).wait()
            rv, ri = proc(lbufa, (NCH - 1) * CT, avals, rv, ri)
            chunk_copy(rg, 1, 0).start()

            def trips(tt, c):
                rvri = list(c)
                rv, ri = rvri[:8], rvri[8:]
                ce = tt * 2
                chunk_copy(rg, ce, 1).wait()
                rv, ri = proc(lbufb, ce * CT, avals, rv, ri)

                @pl.when(ce + 2 <= NCH - 2)
                def _sb():
                    chunk_copy(rg, ce + 2, 1).start()

                co = tt * 2 + 1
                chunk_copy(rg, co, 0).wait()
                rv, ri = proc(lbufa, co * CT, avals, rv, ri)

                @pl.when(co + 2 <= NCH - 2)
                def _sa():
                    chunk_copy(rg, co + 2, 0).start()

                return tuple(rv + ri)

            assert (NCH - 1) % 2 == 0
            out = list(lax.fori_loop(0, (NCH - 1) // 2, trips,
                                     tuple(rv + ri)))
            rv, ri = out[:8], out[8:]

            @pl.when(rg < 3)
            def _prime():
                chunk_copy(rg + 1, NCH - 1, 0).start()
                chunk_copy(rg + 1, 0, 1).start()

            for r in range(8):
                b = rg * 8 + r
                mv = jnp.max(rv[r])
                best = jnp.min(jnp.where(rv[r] == mv, ri[r],
                                         jnp.int32(INT_MAX)))
                lm = iota == (b & (2 * LANES - 1)) % LANES
                lo = b < 16
                m0 = lm & lo
                m1 = lm & (~lo)
                resv0 = jnp.where(m0, mv, resv0)
                resi0 = jnp.where(m0, best, resi0)
                resv1 = jnp.where(m1, mv, resv1)
                resi1 = jnp.where(m1, best, resi1)
            return resv0, resv1, resi0, resi1

        resv0, resv1, resi0, resi1 = lax.fori_loop(
            0, 4, rowgroup, (neg, neg, zero, zero))
        vbuf[pl.ds(0, LANES)] = resv0
        vbuf[pl.ds(LANES, LANES)] = resv1
        ibuf[pl.ds(0, LANES)] = resi0
        ibuf[pl.ds(LANES, LANES)] = resi1
        pltpu.sync_copy(vbuf, pvals_hbm.at[pl.ds(wid * B, B)])
        pltpu.sync_copy(ibuf, pidx_hbm.at[pl.ds(wid * B, B)])

    return sc_sampler, NT * TILE


def kernel(logits, temperatures, exponential):
    B, V = logits.shape
    sampler, vcov = _build_sc_sampler(B, V)
    noise = _tc_log_noise(exponential)
    pv, pi = sampler(logits, temperatures, noise)
    pv = pv.reshape(NW, B)
    pi = pi.reshape(NW, B)
    if vcov < V:
        a = jnp.where(temperatures >= EPS, temperatures, 0.0)
        tk = logits[:, vcov:] - a[:, None] * noise[vcov:]
        tv = jnp.max(tk, axis=-1)
        ti = (vcov + jnp.argmax(tk, axis=-1)).astype(jnp.int32)
        pv = jnp.concatenate([pv, tv[None, :]], axis=0)
        pi = jnp.concatenate([pi, ti[None, :]], axis=0)
    w = jnp.argmax(pv, axis=0)
    out = jnp.take_along_axis(pi, w[None, :], axis=0)[0]
    return out.astype(jnp.int32)
